# baseline (speedup 1.0000x reference)
import functools
import jax
import jax.numpy as jnp
from jax import lax
from jax.experimental import pallas as pl
from jax.experimental.pallas import tpu as pltpu
from jax.experimental.pallas import tpu_sc as plsc

_N = 16384
_NT = 16
_CH = _N // _NT
_NV = _CH // 16
_EPS = 1e-5


def _sc_sort_body(yt_hbm, yp_hbm, s_hbm, ping_k, ping_p, pong_k, pong_p,
                  hall, totals, kloc, ploc, sloc, kt, pt, dloc, dstloc, hist,
                  off_loc, off_flat, lh, hall_loc, t2_loc, sbuf, sem):
    cid = lax.axis_index("c")
    wid = lax.axis_index("s")

    @pl.when(cid == 0)
    def _():
        base = wid * _CH
        lane = lax.iota(jnp.int32, 16)
        lane256 = lane * 256
        iota64 = lane * 64
        zeros16 = jnp.zeros((16,), jnp.int32)

        pltpu.sync_copy(yt_hbm.at[pl.ds(base, _CH)], kloc)
        pltpu.sync_copy(yp_hbm.at[pl.ds(base, _CH)], ploc)

        for p in range(4):
            shift = 8 * p
            if p > 0:
                cur_k_sh = ping_k if (p % 2 == 0) else pong_k
                cur_p_sh = ping_p if (p % 2 == 0) else pong_p
                pltpu.sync_copy(cur_k_sh.at[pl.ds(base, _CH)], kloc)
                pltpu.sync_copy(cur_p_sh.at[pl.ds(base, _CH)], ploc)
            dst_k_sh = pong_k if (p % 2 == 0) else ping_k
            dst_p_sh = pong_p if (p % 2 == 0) else ping_p

            def zero_body(c, _):
                lh[pl.ds(c * 16, 16)] = zeros16
                return 0
            lax.fori_loop(0, 256, zero_body, 0)

            def hist_body(v, _):
                k16 = plsc.load_gather(kloc, [iota64 + v])
                d16 = 255 - ((k16 >> shift) & 255)
                dloc[pl.ds(v * 16, 16)] = d16
                hidx = lane256 + d16
                oldh = plsc.load_gather(lh, [hidx])
                plsc.store_scatter(lh, [hidx], oldh + 1)
                return 0
            lax.fori_loop(0, _NV, hist_body, 0)

            def hred_body(c, _):
                def hred_in(j, acc):
                    return acc + lh[pl.ds(j * 256 + c * 16, 16)]
                hist[pl.ds(c * 16, 16)] = lax.fori_loop(0, _NT, hred_in, zeros16)
                return 0
            lax.fori_loop(0, 16, hred_body, 0)

            pltpu.sync_copy(hist, hall.at[wid])
            plsc.subcore_barrier()
            pltpu.sync_copy(hall, hall_loc)
            plsc.subcore_barrier()

            def scan_body(c, carry):
                def scan_in(t, tp):
                    tot16, part16 = tp
                    h = hall_loc[t, pl.ds(c * 16, 16)]
                    return (tot16 + h,
                            part16 + jnp.where(t < wid, h, 0))
                tot16, part16 = lax.fori_loop(0, _NT, scan_in,
                                              (zeros16, zeros16))
                cs = plsc.cumsum(tot16)
                off_loc[pl.ds(c * 16, 16)] = (cs - tot16) + carry + part16
                return carry + jnp.sum(tot16)
            lax.fori_loop(0, 16, scan_body, jnp.int32(0))

            def lex_body(c, _):
                def lex_in(j, racc):
                    slj = pl.ds(j * 256 + c * 16, 16)
                    off_flat[slj] = racc
                    return racc + lh[slj]
                lax.fori_loop(0, _NT, lex_in, off_loc[pl.ds(c * 16, 16)])
                return 0
            lax.fori_loop(0, 16, lex_body, 0)

            for j8 in range(8):
                def perm_body(vv, _):
                    v = j8 * 8 + vv
                    d16 = dloc[pl.ds(v * 16, 16)]
                    kq = plsc.load_gather(kloc, [iota64 + v])
                    pq = plsc.load_gather(ploc, [iota64 + v])
                    kt[pl.ds(v * 16, 16)] = kq
                    pt[pl.ds(v * 16, 16)] = pq
                    hidx = lane256 + d16
                    dst16 = plsc.load_gather(off_flat, [hidx])
                    plsc.store_scatter(off_flat, [hidx], dst16 + 1)
                    dstloc[j8, pl.ds(vv * 16, 16)] = dst16
                    return 0
                lax.fori_loop(0, 8, perm_body, 0)

            for j in range(8):
                sl = pl.ds(j * 128, 128)
                pltpu.sync_copy(kt.at[sl], dst_k_sh.at[dstloc.at[j]])
                pltpu.sync_copy(pt.at[sl], dst_p_sh.at[dstloc.at[j]])
            plsc.subcore_barrier()

        pltpu.sync_copy(pong_p.at[pl.ds(base, _CH)], ploc)

        def cum_body(v, fcarry):
            sl = pl.ds(v * 16, 16)
            e16 = jnp.exp(ploc[sl])
            incl = plsc.cumsum(e16) + fcarry
            sloc[sl] = incl - e16
            return fcarry + jnp.sum(e16)
        fcarry = lax.fori_loop(0, _NV, cum_body, jnp.float32(0.0))

        sbuf[...] = jnp.zeros((16,), jnp.float32) + fcarry
        pltpu.sync_copy(sbuf, totals.at[wid])
        plsc.subcore_barrier()
        pltpu.sync_copy(totals, t2_loc)

        def tot_body(t, ge):
            grand, excl = ge
            tt = t2_loc[t][0]
            return (grand + tt,
                    excl + jnp.where(t < wid, tt, jnp.float32(0.0)))
        grand, excl = lax.fori_loop(0, _NT, tot_body,
                                    (jnp.float32(0.0), jnp.float32(0.0)))

        def s_body(v, _):
            sl = pl.ds(v * 16, 16)
            sloc[sl] = grand - (excl + sloc[sl])
            return 0
        lax.fori_loop(0, _NV, s_body, 0)
        pltpu.sync_copy(sloc, s_hbm.at[pl.ds(base, _CH)])


@functools.partial(
    pl.kernel,
    out_type=jax.ShapeDtypeStruct((_N,), jnp.float32),
    mesh=plsc.VectorSubcoreMesh(core_axis_name="c", subcore_axis_name="s"),
    compiler_params=pltpu.CompilerParams(needs_layout_passes=False),
    scratch_types=[
        pltpu.VMEM_SHARED((_N,), jnp.int32),
        pltpu.VMEM_SHARED((_N,), jnp.float32),
        pltpu.VMEM_SHARED((_N,), jnp.int32),
        pltpu.VMEM_SHARED((_N,), jnp.float32),
        pltpu.VMEM_SHARED((_NT, 256), jnp.int32),
        pltpu.VMEM_SHARED((_NT, 16), jnp.float32),
        pltpu.VMEM((_CH,), jnp.int32),
        pltpu.VMEM((_CH,), jnp.float32),
        pltpu.VMEM((_CH,), jnp.float32),
        pltpu.VMEM((_CH,), jnp.int32),
        pltpu.VMEM((_CH,), jnp.float32),
        pltpu.VMEM((_CH,), jnp.int32),
        pltpu.VMEM((8, 128), jnp.int32),
        pltpu.VMEM((256,), jnp.int32),
        pltpu.VMEM((256,), jnp.int32),
        pltpu.VMEM((16 * 256,), jnp.int32),
        pltpu.VMEM((16 * 256,), jnp.int32),
        pltpu.VMEM((_NT, 256), jnp.int32),
        pltpu.VMEM((_NT, 16), jnp.float32),
        pltpu.VMEM((16,), jnp.float32),
        pltpu.SemaphoreType.DMA,
    ],
)
def _sc_sort(yt_hbm, yp_hbm, s_hbm, *scratch):
    _sc_sort_body(yt_hbm, yp_hbm, s_hbm, *scratch)


def _tc_reduce_body(s_ref, yp_ref, out_ref):
    out_ref[...] = (jnp.sum(jnp.log(s_ref[...] + _EPS))
                    - jnp.sum(yp_ref[...])).reshape(1, 1)


def kernel(y_pred, y_true):
    ut = lax.bitcast_convert_type(y_true, jnp.int32)
    s = _sc_sort(ut, y_pred)
    out = pl.pallas_call(
        _tc_reduce_body,
        out_shape=jax.ShapeDtypeStruct((1, 1), jnp.float32),
    )(s.reshape(128, 128), y_pred.reshape(128, 128))
    return out[0, 0]


# speedup vs baseline: 8.2943x; 8.2943x over previous
import jax
import jax.numpy as jnp
from jax import lax
from jax.experimental import pallas as pl
from jax.experimental.pallas import tpu as pltpu

_N = 16384
_R = 128
_C = 128
_EPS = 1e-5


def _listmle_body(yp_ref, yt_ref, out_ref):
    yt = yt_ref[...]
    yp = yp_ref[...]
    u = lax.bitcast_convert_type(yt, jnp.int32)
    row = lax.broadcasted_iota(jnp.int32, (_R, _C), 0)
    col = lax.broadcasted_iota(jnp.int32, (_R, _C), 1)
    idx = row * _C + col
    ypbits = lax.bitcast_convert_type(yp, jnp.uint32)
    packed = ((16383 - idx).astype(jnp.uint32) << 18) | ((ypbits + 0x2000) >> 14)

    def exchange(ku, vv, pu, pv, bit):
        cmp = (ku < pu) | ((ku == pu) & (vv < pv))
        sel = cmp ^ bit
        return jnp.where(sel, ku, pu), jnp.where(sel, vv, pv)

    def free_swap(x, g):
        rr = x.shape[0]
        pieces = []
        for j in range(0, rr, 2 * g):
            pieces.append(lax.slice_in_dim(x, j + g, j + 2 * g, axis=0))
            pieces.append(lax.slice_in_dim(x, j, j + g, axis=0))
        return jnp.concatenate(pieces, axis=0)

    def sublane_free_stage(ku, vv, g, rows):
        pu, pv = free_swap(ku, g), free_swap(vv, g)
        return exchange(ku, vv, pu, pv, (rows & g) != 0)

    _RS = 32
    col_s = lax.broadcasted_iota(jnp.int32, (_RS, _C), 1)
    row_s = lax.broadcasted_iota(jnp.int32, (_RS, _C), 0)

    def lane_cascade(ku, vv, g_top):
        for g_exp in range(g_top.bit_length() - 1, -1, -1):
            g = 1 << g_exp
            bitg = (col_s & g) != 0
            pu = jnp.where(bitg, pltpu.roll(ku, g, 1), pltpu.roll(ku, _C - g, 1))
            pv = jnp.where(bitg, pltpu.roll(vv, g, 1), pltpu.roll(vv, _C - g, 1))
            ku, vv = exchange(ku, vv, pu, pv, bitg)
        return ku, vv

    def low_cascade(ku, vv, d_top):
        for d_exp in range(d_top.bit_length() - 1, -1, -1):
            d = 1 << d_exp
            if d >= 8:
                ku, vv = sublane_free_stage(ku, vv, d, row_s)
            else:
                bitd = (row_s & d) != 0
                pu = jnp.where(bitd, pltpu.roll(ku, d, 0),
                               pltpu.roll(ku, _RS - d, 0))
                pv = jnp.where(bitd, pltpu.roll(vv, d, 0),
                               pltpu.roll(vv, _RS - d, 0))
                ku, vv = exchange(ku, vv, pu, pv, bitd)
        return ku, vv

    def over_slabs(ku, vv, fn):
        slabs = []
        for s in range(0, _R, _RS):
            slabs.append(fn(lax.slice_in_dim(ku, s, s + _RS, axis=0),
                            lax.slice_in_dim(vv, s, s + _RS, axis=0)))
        return (jnp.concatenate([a for a, _ in slabs], axis=0),
                jnp.concatenate([b for _, b in slabs], axis=0))

    key_u = lax.bitcast_convert_type(u, jnp.uint32)
    val = packed
    flat = col * _R + row
    dm2 = jnp.where((flat & 2) != 0, jnp.uint32(0xFFFFFFFF), jnp.uint32(0))
    key_u = key_u ^ dm2
    val = val ^ dm2
    for k_exp in range(1, 15):
        k = 1 << k_exp
        if k_exp - 1 >= 7:
            gt = min(k // 2, _N // 2) // _R
            key_u, val = over_slabs(key_u, val,
                                    lambda a, b: lane_cascade(a, b, gt))
        for g in (64, 32):
            if g <= k // 2:
                key_u, val = sublane_free_stage(key_u, val, g, row)
        dt = min(k // 2, 16)
        key_u, val = over_slabs(key_u, val,
                                lambda a, b: low_cascade(a, b, dt))
        if k < _N:
            nk = 2 * k
            if nk < _N:
                m = ((flat & k) != 0) ^ ((flat & nk) != 0)
            else:
                m = (flat & k) != 0
            dmc = jnp.where(m, jnp.uint32(0xFFFFFFFF), jnp.uint32(0))
            key_u = key_u ^ dmc
            val = val ^ dmc

    e = jnp.exp(lax.bitcast_convert_type(val << 14, jnp.float32))
    acc = e
    for d in (1, 2, 4, 8, 16, 32, 64):
        acc = acc + jnp.where(row >= d, jnp.roll(acc, d, axis=0), 0.0)
    cs = jnp.sum(e, axis=0, keepdims=True)
    col1 = lax.broadcasted_iota(jnp.int32, (1, _C), 1)
    cacc = cs
    for d in (1, 2, 4, 8, 16, 32, 64):
        cacc = cacc + jnp.where(col1 >= d, jnp.roll(cacc, d, axis=1), 0.0)
    p = acc + (cacc - cs)
    total = jnp.sum(jnp.log(p + _EPS)) - jnp.sum(yp)
    out_ref[...] = total.reshape(1, 1)


def kernel(y_pred, y_true):
    yp = y_pred.reshape(_R, _C)
    yt = y_true.reshape(_R, _C)
    out = pl.pallas_call(
        _listmle_body,
        out_shape=jax.ShapeDtypeStruct((1, 1), jnp.float32),
    )(yp, yt)
    return out[0, 0]


# speedup vs baseline: 8.5732x; 1.0336x over previous
import jax
import jax.numpy as jnp
from jax import lax
from jax.experimental import pallas as pl
from jax.experimental.pallas import tpu as pltpu

_N = 16384
_R = 128
_C = 128
_EPS = 1e-5


def _listmle_body(yp_ref, yt_ref, out_ref):
    yt = yt_ref[...]
    yp = yp_ref[...]
    u = lax.bitcast_convert_type(yt, jnp.int32)
    row = lax.broadcasted_iota(jnp.int32, (_R, _C), 0)
    col = lax.broadcasted_iota(jnp.int32, (_R, _C), 1)
    idx = row * _C + col
    ypbits = lax.bitcast_convert_type(yp, jnp.uint32)
    packed = ((16383 - idx).astype(jnp.uint32) << 18) | ((ypbits + 0x2000) >> 14)

    def exchange(ku, vv, pu, pv, bit):
        cmp = (ku < pu) | ((ku == pu) & (vv < pv))
        sel = cmp ^ bit
        return jnp.where(sel, ku, pu), jnp.where(sel, vv, pv)

    def free_swap(x, g):
        rr = x.shape[0]
        pieces = []
        for j in range(0, rr, 2 * g):
            pieces.append(lax.slice_in_dim(x, j + g, j + 2 * g, axis=0))
            pieces.append(lax.slice_in_dim(x, j, j + g, axis=0))
        return jnp.concatenate(pieces, axis=0)

    def sublane_free_stage(ku, vv, g, rows):
        pu, pv = free_swap(ku, g), free_swap(vv, g)
        return exchange(ku, vv, pu, pv, (rows & g) != 0)

    _RS = 32
    _RSL = 16
    col_s = lax.broadcasted_iota(jnp.int32, (_RSL, _C), 1)
    row_s = lax.broadcasted_iota(jnp.int32, (_RS, _C), 0)

    def lane_cascade(ku, vv, g_top):
        for g_exp in range(g_top.bit_length() - 1, -1, -1):
            g = 1 << g_exp
            bitg = (col_s & g) != 0
            pu = jnp.where(bitg, pltpu.roll(ku, g, 1), pltpu.roll(ku, _C - g, 1))
            pv = jnp.where(bitg, pltpu.roll(vv, g, 1), pltpu.roll(vv, _C - g, 1))
            ku, vv = exchange(ku, vv, pu, pv, bitg)
        return ku, vv

    def low_cascade(ku, vv, d_top):
        for d_exp in range(d_top.bit_length() - 1, -1, -1):
            d = 1 << d_exp
            if d >= 8:
                ku, vv = sublane_free_stage(ku, vv, d, row_s)
            else:
                bitd = (row_s & d) != 0
                pu = jnp.where(bitd, pltpu.roll(ku, d, 0),
                               pltpu.roll(ku, _RS - d, 0))
                pv = jnp.where(bitd, pltpu.roll(vv, d, 0),
                               pltpu.roll(vv, _RS - d, 0))
                ku, vv = exchange(ku, vv, pu, pv, bitd)
        return ku, vv

    def over_slabs(ku, vv, fn, rs):
        slabs = []
        for s in range(0, _R, rs):
            slabs.append(fn(lax.slice_in_dim(ku, s, s + rs, axis=0),
                            lax.slice_in_dim(vv, s, s + rs, axis=0)))
        return (jnp.concatenate([a for a, _ in slabs], axis=0),
                jnp.concatenate([b for _, b in slabs], axis=0))

    key_u = lax.bitcast_convert_type(u, jnp.uint32)
    val = packed
    flat = col * _R + row
    dm2 = jnp.where((flat & 2) != 0, jnp.uint32(0xFFFFFFFF), jnp.uint32(0))
    key_u = key_u ^ dm2
    val = val ^ dm2
    for k_exp in range(1, 15):
        k = 1 << k_exp
        if k_exp - 1 >= 7:
            gt = min(k // 2, _N // 2) // _R
            key_u, val = over_slabs(key_u, val,
                                    lambda a, b: lane_cascade(a, b, gt), _RSL)
        for g in (64, 32):
            if g <= k // 2:
                key_u, val = sublane_free_stage(key_u, val, g, row)
        dt = min(k // 2, 16)
        key_u, val = over_slabs(key_u, val,
                                lambda a, b: low_cascade(a, b, dt), _RS)
        if k < _N:
            nk = 2 * k
            if nk < _N:
                m = ((flat & k) != 0) ^ ((flat & nk) != 0)
            else:
                m = (flat & k) != 0
            dmc = jnp.where(m, jnp.uint32(0xFFFFFFFF), jnp.uint32(0))
            key_u = key_u ^ dmc
            val = val ^ dmc

    e = jnp.exp(lax.bitcast_convert_type(val << 14, jnp.float32))
    acc = e
    for d in (1, 2, 4, 8, 16, 32, 64):
        acc = acc + jnp.where(row >= d, jnp.roll(acc, d, axis=0), 0.0)
    cs = jnp.sum(e, axis=0, keepdims=True)
    col1 = lax.broadcasted_iota(jnp.int32, (1, _C), 1)
    cacc = cs
    for d in (1, 2, 4, 8, 16, 32, 64):
        cacc = cacc + jnp.where(col1 >= d, jnp.roll(cacc, d, axis=1), 0.0)
    p = acc + (cacc - cs)
    total = jnp.sum(jnp.log(p + _EPS)) - jnp.sum(yp)
    out_ref[...] = total.reshape(1, 1)


def kernel(y_pred, y_true):
    yp = y_pred.reshape(_R, _C)
    yt = y_true.reshape(_R, _C)
    out = pl.pallas_call(
        _listmle_body,
        out_shape=jax.ShapeDtypeStruct((1, 1), jnp.float32),
    )(yp, yt)
    return out[0, 0]


# speedup vs baseline: 8.7704x; 1.0230x over previous
import jax
import jax.numpy as jnp
from jax import lax
from jax.experimental import pallas as pl
from jax.experimental.pallas import tpu as pltpu

_N = 16384
_R = 128
_C = 128
_EPS = 1e-5


def _listmle_body(yp_ref, yt_ref, out_ref):
    yt = yt_ref[...]
    yp = yp_ref[...]
    u = lax.bitcast_convert_type(yt, jnp.int32)
    row = lax.broadcasted_iota(jnp.int32, (_R, _C), 0)
    col = lax.broadcasted_iota(jnp.int32, (_R, _C), 1)
    idx = row * _C + col
    ypbits = lax.bitcast_convert_type(yp, jnp.uint32)
    packed = ((16383 - idx).astype(jnp.uint32) << 18) | ((ypbits + 0x2000) >> 14)

    def exchange(ku, vv, pu, pv, bit):
        cmp = (ku < pu) | ((ku == pu) & (vv < pv))
        sel = cmp ^ bit
        return jnp.where(sel, ku, pu), jnp.where(sel, vv, pv)

    def free_swap(x, g):
        rr = x.shape[0]
        pieces = []
        for j in range(0, rr, 2 * g):
            pieces.append(lax.slice_in_dim(x, j + g, j + 2 * g, axis=0))
            pieces.append(lax.slice_in_dim(x, j, j + g, axis=0))
        return jnp.concatenate(pieces, axis=0)

    def sublane_free_stage(ku, vv, g, rows):
        pu, pv = free_swap(ku, g), free_swap(vv, g)
        return exchange(ku, vv, pu, pv, (rows & g) != 0)

    _RS = 32
    _RSL = 8
    col_s = lax.broadcasted_iota(jnp.int32, (_RSL, _C), 1)
    row_s = lax.broadcasted_iota(jnp.int32, (_RS, _C), 0)

    def lane_cascade(ku, vv, g_top):
        for g_exp in range(g_top.bit_length() - 1, -1, -1):
            g = 1 << g_exp
            bitg = (col_s & g) != 0
            pu = jnp.where(bitg, pltpu.roll(ku, g, 1), pltpu.roll(ku, _C - g, 1))
            pv = jnp.where(bitg, pltpu.roll(vv, g, 1), pltpu.roll(vv, _C - g, 1))
            ku, vv = exchange(ku, vv, pu, pv, bitg)
        return ku, vv

    def low_cascade(ku, vv, d_top):
        for d_exp in range(d_top.bit_length() - 1, -1, -1):
            d = 1 << d_exp
            if d >= 8:
                ku, vv = sublane_free_stage(ku, vv, d, row_s)
            else:
                bitd = (row_s & d) != 0
                pu = jnp.where(bitd, pltpu.roll(ku, d, 0),
                               pltpu.roll(ku, _RS - d, 0))
                pv = jnp.where(bitd, pltpu.roll(vv, d, 0),
                               pltpu.roll(vv, _RS - d, 0))
                ku, vv = exchange(ku, vv, pu, pv, bitd)
        return ku, vv

    def over_slabs(ku, vv, fn, rs):
        slabs = []
        for s in range(0, _R, rs):
            slabs.append(fn(lax.slice_in_dim(ku, s, s + rs, axis=0),
                            lax.slice_in_dim(vv, s, s + rs, axis=0)))
        return (jnp.concatenate([a for a, _ in slabs], axis=0),
                jnp.concatenate([b for _, b in slabs], axis=0))

    key_u = lax.bitcast_convert_type(u, jnp.uint32)
    val = packed
    flat = col * _R + row
    dm2 = jnp.where((flat & 2) != 0, jnp.uint32(0xFFFFFFFF), jnp.uint32(0))
    key_u = key_u ^ dm2
    val = val ^ dm2
    for k_exp in range(1, 15):
        k = 1 << k_exp
        if k_exp - 1 >= 7:
            gt = min(k // 2, _N // 2) // _R
            key_u, val = over_slabs(key_u, val,
                                    lambda a, b: lane_cascade(a, b, gt), _RSL)
        for g in (64, 32):
            if g <= k // 2:
                key_u, val = sublane_free_stage(key_u, val, g, row)
        dt = min(k // 2, 16)
        key_u, val = over_slabs(key_u, val,
                                lambda a, b: low_cascade(a, b, dt), _RS)
        if k < _N:
            nk = 2 * k
            if nk < _N:
                m = ((flat & k) != 0) ^ ((flat & nk) != 0)
            else:
                m = (flat & k) != 0
            dmc = jnp.where(m, jnp.uint32(0xFFFFFFFF), jnp.uint32(0))
            key_u = key_u ^ dmc
            val = val ^ dmc

    e = jnp.exp(lax.bitcast_convert_type(val << 14, jnp.float32))
    acc = e
    for d in (1, 2, 4, 8, 16, 32, 64):
        acc = acc + jnp.where(row >= d, jnp.roll(acc, d, axis=0), 0.0)
    cs = jnp.sum(e, axis=0, keepdims=True)
    col1 = lax.broadcasted_iota(jnp.int32, (1, _C), 1)
    cacc = cs
    for d in (1, 2, 4, 8, 16, 32, 64):
        cacc = cacc + jnp.where(col1 >= d, jnp.roll(cacc, d, axis=1), 0.0)
    p = acc + (cacc - cs)
    total = jnp.sum(jnp.log(p + _EPS)) - jnp.sum(yp)
    out_ref[...] = total.reshape(1, 1)


def kernel(y_pred, y_true):
    yp = y_pred.reshape(_R, _C)
    yt = y_true.reshape(_R, _C)
    out = pl.pallas_call(
        _listmle_body,
        out_shape=jax.ShapeDtypeStruct((1, 1), jnp.float32),
    )(yp, yt)
    return out[0, 0]
